# baseline (speedup 1.0000x reference)
import jax
import jax.numpy as jnp
from jax.experimental import pallas as pl
from jax.experimental.pallas import tpu as pltpu

HASH_DIM = 8
N_HASHES = 4
BLK = 4096


def _fused_kernel(x_ref, w_ref, rot_ref, u_ref, sim_ref, ort_ref):
    w = w_ref[...]
    uvec = u_ref[0, :]
    uh = uvec / (jnp.sqrt(jnp.sum(uvec * uvec)) + 1e-6)
    hm = jnp.eye(HASH_DIM, dtype=jnp.float32) - 2.0 * uh[:, None] * uh[None, :]
    cats = [rot_ref[h] for h in range(N_HASHES)]
    cats += [rot_ref[h] @ hm for h in range(N_HASHES)]
    ccat = jax.lax.dot_general(
        w, jnp.concatenate(cats, axis=1),
        dimension_numbers=(((0,), (0,)), ((), ())),
        preferred_element_type=jnp.float32)
    out = jnp.dot(x_ref[...], ccat, preferred_element_type=jnp.float32)
    q = BLK // 4
    junk = jnp.concatenate([out[:q, :], out[q:2 * q, :]], axis=1)
    sim_ref[...] = junk
    ort_ref[...] = junk


def kernel(x, W_base, rot, u):
    B, N, D = x.shape
    T = B * N
    x2 = x.reshape(T, D)

    sim2, ort2 = pl.pallas_call(
        _fused_kernel,
        grid=(T // BLK,),
        in_specs=[
            pl.BlockSpec((BLK, D), lambda i: (i, 0)),
            pl.BlockSpec((HASH_DIM, D), lambda i: (0, 0)),
            pl.BlockSpec((N_HASHES, HASH_DIM, HASH_DIM), lambda i: (0, 0, 0)),
            pl.BlockSpec((1, HASH_DIM), lambda i: (0, 0)),
        ],
        out_specs=[
            pl.BlockSpec((BLK // 4, 128), lambda i: (i, 0)),
            pl.BlockSpec((BLK // 4, 128), lambda i: (i, 0)),
        ],
        out_shape=[
            jax.ShapeDtypeStruct((T // 4, 128), jnp.float32),
            jax.ShapeDtypeStruct((T // 4, 128), jnp.float32),
        ],
        compiler_params=pltpu.CompilerParams(
            dimension_semantics=("arbitrary",)),
    )(x2, W_base, rot, u.reshape(1, HASH_DIM))

    sim = sim2.reshape(B, N, N_HASHES, HASH_DIM)
    ort = ort2.reshape(B, N, N_HASHES, HASH_DIM)
    return (sim, ort)


# speedup vs baseline: 3.1882x; 3.1882x over previous
import jax
import jax.numpy as jnp
from jax.experimental import pallas as pl
from jax.experimental.pallas import tpu as pltpu

HASH_DIM = 8
N_HASHES = 4
K_STREAMS = 8
BLK = 512


def _fused_kernel(*refs):
    x_refs = refs[:K_STREAMS]
    w_ref, rot_ref, u_ref, sim_ref, ort_ref = refs[K_STREAMS:]
    w = w_ref[...]
    uvec = u_ref[0, :]
    uh = uvec / (jnp.sqrt(jnp.sum(uvec * uvec)) + 1e-6)
    hm = jnp.eye(HASH_DIM, dtype=jnp.float32) - 2.0 * uh[:, None] * uh[None, :]
    cats = [rot_ref[h] for h in range(N_HASHES)]
    cats += [rot_ref[h] @ hm for h in range(N_HASHES)]
    ccat = jax.lax.dot_general(
        w, jnp.concatenate(cats, axis=1),
        dimension_numbers=(((0,), (0,)), ((), ())),
        preferred_element_type=jnp.float32)
    half = N_HASHES * HASH_DIM
    for j in range(K_STREAMS):
        out = jnp.dot(x_refs[j][...], ccat, preferred_element_type=jnp.float32)
        sim_ref[j * BLK:(j + 1) * BLK, :] = out[:, :half]
        ort_ref[j * BLK:(j + 1) * BLK, :] = out[:, half:]


def kernel(x, W_base, rot, u):
    B, N, D = x.shape
    T = B * N
    x2 = x.reshape(T, D)
    cols = N_HASHES * HASH_DIM
    step_rows = K_STREAMS * BLK

    def x_spec(j):
        return pl.BlockSpec((BLK, D), lambda i, j=j: (i * K_STREAMS + j, 0))

    sim2, ort2 = pl.pallas_call(
        _fused_kernel,
        grid=(T // step_rows,),
        in_specs=[x_spec(j) for j in range(K_STREAMS)] + [
            pl.BlockSpec((HASH_DIM, D), lambda i: (0, 0)),
            pl.BlockSpec((N_HASHES, HASH_DIM, HASH_DIM), lambda i: (0, 0, 0)),
            pl.BlockSpec((1, HASH_DIM), lambda i: (0, 0)),
        ],
        out_specs=[
            pl.BlockSpec((step_rows, cols), lambda i: (i, 0)),
            pl.BlockSpec((step_rows, cols), lambda i: (i, 0)),
        ],
        out_shape=[
            jax.ShapeDtypeStruct((T, cols), jnp.float32),
            jax.ShapeDtypeStruct((T, cols), jnp.float32),
        ],
        compiler_params=pltpu.CompilerParams(
            dimension_semantics=("arbitrary",)),
    )(*([x2] * K_STREAMS), W_base, rot, u.reshape(1, HASH_DIM))

    sim = sim2.reshape(B, N, N_HASHES, HASH_DIM)
    ort = ort2.reshape(B, N, N_HASHES, HASH_DIM)
    return (sim, ort)
